# Initial kernel scaffold; baseline (speedup 1.0000x reference)
import functools

import jax
import jax.numpy as jnp
from jax import lax
from jax.experimental import pallas as pl
from jax.experimental.pallas import tpu as pltpu
from jax.experimental.pallas import tpu_sc as plsc

B, L = 16384, 200
N = B * L
NC, NS, LANES = 2, 16, 16
NW = NC * NS
CPW = N // NW
C = 2048
NCHUNK = CPW // C
G = C // 128


@functools.partial(
    pl.kernel,
    out_type=jax.ShapeDtypeStruct((N * 3,), jnp.float32),
    mesh=plsc.VectorSubcoreMesh(core_axis_name="c", subcore_axis_name="s"),
    scratch_types=[
        pltpu.VMEM((C,), jnp.int32),
        pltpu.VMEM((C,), jnp.float32),
        pltpu.VMEM((3 * C,), jnp.float32),
        pltpu.VMEM((16,), jnp.float32),
        pltpu.SemaphoreType.DMA,
    ],
)
def _gather_linear(idx_hbm, tab_hbm, wb_hbm, out_hbm,
                   idx_v, vals_v, out3_v, wb_v, sem):
    cid = lax.axis_index("c")
    sid = lax.axis_index("s")
    wid = sid * NC + cid
    base0 = wid * CPW

    pltpu.sync_copy(wb_hbm, wb_v)
    ii = lax.broadcasted_iota(jnp.int32, (16,), 0)
    npat, wpat, bpat = [], [], []
    for r in range(3):
        m = ii + 16 * r
        kk = m % 3
        npat.append(m // 3)
        wpat.append(plsc.load_gather(wb_v, [kk]))
        bpat.append(plsc.load_gather(wb_v, [kk + 3]))

    def chunk_body(t, carry):
        base = base0 + t * C
        pltpu.sync_copy(idx_hbm.at[pl.ds(base, C)], idx_v)
        copies = [
            pltpu.async_copy(
                tab_hbm.at[idx_v.at[pl.ds(g * 128, 128)]],
                vals_v.at[pl.ds(g * 128, 128)],
                sem,
            )
            for g in range(G)
        ]
        for cp in copies:
            cp.wait()

        def j_body(j, _):
            nb = j * 16
            for r in range(3):
                v = plsc.load_gather(vals_v, [npat[r] + nb])
                out3_v[pl.ds(j * 48 + r * 16, 16)] = v * wpat[r] + bpat[r]
            return 0

        lax.fori_loop(0, C // 16, j_body, 0)
        pltpu.sync_copy(out3_v, out_hbm.at[pl.ds(3 * base, 3 * C)])
        return carry

    lax.fori_loop(0, NCHUNK, chunk_body, 0)


def kernel(data, table, W, b):
    idx = data.reshape(-1)
    tab = table.reshape(-1)
    wb = jnp.concatenate(
        [W.reshape(-1), b.reshape(-1), jnp.zeros((10,), jnp.float32)]
    )
    out_flat = _gather_linear(idx, tab, wb)
    return out_flat.reshape(B, L, 3)


# speedup vs baseline: 2.4054x; 2.4054x over previous
import functools

import jax
import jax.numpy as jnp
import numpy as np
from jax import lax
from jax.experimental import pallas as pl
from jax.experimental.pallas import tpu as pltpu
from jax.experimental.pallas import tpu_sc as plsc

B, L = 16384, 200
N = B * L
V = 1_000_000
VP = 1 << 20
NC, NS = 2, 16
NW = NC * NS

CPW = N // NW
C = 2048
NCHUNK = CPW // C
SW = 512
G = C // SW

VPW = VP // NW
CV = 2048
NVCHUNK = VPW // CV


SB = 512


def _expand_body(w_ref, b_ref, tab_ref, out_ref):
    t = tab_ref[...]
    for k in range(3):
        out_ref[k] = t * w_ref[k] + b_ref[k]


def _expand_planes(tab2d, W3, b3):
    return pl.pallas_call(
        _expand_body,
        grid=(VP // 128 // SB,),
        in_specs=[
            pl.BlockSpec(memory_space=pltpu.SMEM),
            pl.BlockSpec(memory_space=pltpu.SMEM),
            pl.BlockSpec((SB, 128), lambda i: (i, 0)),
        ],
        out_specs=pl.BlockSpec((3, SB, 128), lambda i: (0, i, 0)),
        out_shape=jax.ShapeDtypeStruct((3, VP // 128, 128), jnp.float32),
    )(W3, b3, tab2d)


@functools.partial(
    pl.kernel,
    out_type=jax.ShapeDtypeStruct((VP, 8), jnp.float32),
    mesh=plsc.VectorSubcoreMesh(core_axis_name="c", subcore_axis_name="s"),
    compiler_params=pltpu.CompilerParams(
        needs_layout_passes=False, use_tc_tiling_on_sc=False
    ),
    scratch_types=[pltpu.VMEM((CV, 8), jnp.float32)],
)
def _interleave(planes_hbm, out_hbm, buf8_v):
    cid = lax.axis_index("c")
    sid = lax.axis_index("s")
    wid = sid * NC + cid
    vbase0 = wid * VPW

    @pl.loop(0, NVCHUNK)
    def chunk(t):
        vbase = vbase0 + t * CV
        for k in range(3):
            pltpu.sync_copy(
                planes_hbm.at[pl.ds(k * VP + vbase, CV)],
                buf8_v.at[:, pl.ds(k, 1)],
            )
        pltpu.sync_copy(buf8_v, out_hbm.at[pl.ds(vbase, CV)])


@functools.partial(
    pl.kernel,
    out_type=jax.ShapeDtypeStruct((N, 3), jnp.float32),
    mesh=plsc.VectorSubcoreMesh(core_axis_name="c", subcore_axis_name="s"),
    compiler_params=pltpu.CompilerParams(
        needs_layout_passes=False, use_tc_tiling_on_sc=False
    ),
    scratch_types=[
        pltpu.VMEM((C,), jnp.int32),
        pltpu.VMEM((C, 8), jnp.float32),
        pltpu.SemaphoreType.DMA,
    ],
)
def _row_gather(idx_hbm, t8_hbm, out_hbm, idx_v, vals_v, sem):
    cid = lax.axis_index("c")
    sid = lax.axis_index("s")
    wid = sid * NC + cid
    base0 = wid * CPW

    @pl.loop(0, NCHUNK)
    def chunk(t):
        base = base0 + t * C
        pltpu.sync_copy(idx_hbm.at[pl.ds(base, C)], idx_v)
        copies = [
            pltpu.async_copy(
                t8_hbm.at[idx_v.at[pl.ds(g * SW, SW)]],
                vals_v.at[pl.ds(g * SW, SW)],
                sem,
            )
            for g in range(G)
        ]
        for cp in copies:
            cp.wait()
        pltpu.sync_copy(vals_v.at[:, pl.ds(0, 3)], out_hbm.at[pl.ds(base, C)])


def kernel(data, table, W, b):
    idx = data.reshape(-1)
    tab = jnp.pad(table.reshape(-1), (0, VP - V)).reshape(VP // 128, 128)
    planes = _expand_planes(tab, W.reshape(-1), b.reshape(-1))
    t8 = _interleave(planes.reshape(-1, 1))
    out = _row_gather(idx, t8)
    return out.reshape(B, L, 3)
